# Initial kernel scaffold; baseline (speedup 1.0000x reference)
import jax
import jax.numpy as jnp
from jax.experimental import pallas as pl


def kernel(xin, codebooks):
    raise NotImplementedError("write your pallas kernel here")


# speedup vs baseline: 6.2309x; 6.2309x over previous
import functools

import jax
import jax.numpy as jnp
from jax.experimental import pallas as pl
from jax.experimental.pallas import tpu as pltpu

_G = 4
_K = 160


def _vq_kernel(x_ref, cb_ref, zq_ref, codes_ref, loss_ref, *, n_total):
    b = pl.program_id(0)
    tt = pl.program_id(1)

    @pl.when(jnp.logical_and(b == 0, tt == 0))
    def _init():
        loss_ref[0, 0] = jnp.float32(0.0)

    x = x_ref[0]
    dg = cb_ref.shape[2]
    t_w = x.shape[1]
    loss_tile = jnp.float32(0.0)
    mi_rows = []
    for g in range(_G):
        xg = x[g * dg:(g + 1) * dg, :]
        cb = cb_ref[g]
        cb2 = jnp.sum(cb * cb, axis=1)
        x2 = jnp.sum(xg * xg, axis=0)
        m = jnp.dot(cb.astype(jnp.bfloat16), xg.astype(jnp.bfloat16),
                    preferred_element_type=jnp.float32)
        d = (x2[None, :] + cb2[:, None]) - 2.0 * m
        iota_k = jax.lax.broadcasted_iota(jnp.int32, (_K, t_w), 0)
        dmin = jnp.min(d, axis=0)
        mi = jnp.min(jnp.where(d == dmin[None, :], iota_k, _K),
                     axis=0).astype(jnp.int32)
        onehot = (iota_k == mi[None, :]).astype(jnp.float32)
        zq_g = jax.lax.dot_general(
            cb, onehot, (((0,), (0,)), ((), ())),
            precision=jax.lax.Precision.HIGHEST,
            preferred_element_type=jnp.float32)
        zq_ref[0, g * dg:(g + 1) * dg, :] = zq_g
        mi_rows.append(mi)
        loss_tile += jnp.sum((zq_g - xg) ** 2)

    codes_ref[0] = jnp.stack(mi_rows, axis=0)
    loss_ref[0, 0] += loss_tile * (1.25 / n_total)


@jax.jit
def kernel(xin, codebooks):
    B, C, T = xin.shape
    G, K, dg = codebooks.shape
    t_blk = 1024 if T % 1024 == 0 else T
    grid = (B, T // t_blk)

    zq, codes, loss = pl.pallas_call(
        functools.partial(_vq_kernel, n_total=B * C * T),
        grid=grid,
        in_specs=[
            pl.BlockSpec((1, C, t_blk), lambda b, t: (b, 0, t)),
            pl.BlockSpec((G, K, dg), lambda b, t: (0, 0, 0)),
        ],
        out_specs=[
            pl.BlockSpec((1, C, t_blk), lambda b, t: (b, 0, t)),
            pl.BlockSpec((1, G, t_blk), lambda b, t: (b, 0, t)),
            pl.BlockSpec((1, 1), lambda b, t: (0, 0),
                         memory_space=pltpu.SMEM),
        ],
        out_shape=[
            jax.ShapeDtypeStruct((B, C, T), jnp.float32),
            jax.ShapeDtypeStruct((B, G, T), jnp.int32),
            jax.ShapeDtypeStruct((1, 1), jnp.float32),
        ],
    )(xin, codebooks)
    return zq, loss[0, 0], codes


# speedup vs baseline: 10.3376x; 1.6591x over previous
import functools

import jax
import jax.numpy as jnp
from jax.experimental import pallas as pl
from jax.experimental.pallas import tpu as pltpu

_G = 4
_K = 160


def _vq_kernel(x_ref, cb_ref, zq_ref, codes_ref, loss_ref, *, n_total):
    b = pl.program_id(0)
    tt = pl.program_id(1)

    @pl.when(jnp.logical_and(b == 0, tt == 0))
    def _init():
        loss_ref[0, 0] = jnp.float32(0.0)

    x = x_ref[0]
    dg = cb_ref.shape[2]
    t_w = x.shape[1]
    loss_tile = jnp.float32(0.0)
    mi_rows = []
    for g in range(_G):
        xg = x[g * dg:(g + 1) * dg, :]
        cb = cb_ref[g]
        cb2 = jnp.sum(cb * cb, axis=1)
        x2 = jnp.sum(xg * xg, axis=0)
        m = jnp.dot(cb.astype(jnp.bfloat16), xg.astype(jnp.bfloat16),
                    preferred_element_type=jnp.float32)
        d = (x2[None, :] + cb2[:, None]) - 2.0 * m
        iota_k = jax.lax.broadcasted_iota(jnp.int32, (_K, t_w), 0)
        dmin = jnp.min(d, axis=0)
        mi = jnp.min(jnp.where(d == dmin[None, :], iota_k, _K),
                     axis=0).astype(jnp.int32)
        onehot = (iota_k == mi[None, :]).astype(jnp.bfloat16)
        cb_hi = cb.astype(jnp.bfloat16)
        cb_lo = (cb - cb_hi.astype(jnp.float32)).astype(jnp.bfloat16)
        dims = (((0,), (0,)), ((), ()))
        zq_g = (jax.lax.dot_general(cb_hi, onehot, dims,
                                    preferred_element_type=jnp.float32)
                + jax.lax.dot_general(cb_lo, onehot, dims,
                                      preferred_element_type=jnp.float32))
        zq_ref[0, g * dg:(g + 1) * dg, :] = zq_g
        mi_rows.append(mi)
        loss_tile += jnp.sum(dmin)

    codes_ref[0] = jnp.stack(mi_rows, axis=0)
    loss_ref[0, 0] += loss_tile * (1.25 / n_total)


@jax.jit
def kernel(xin, codebooks):
    B, C, T = xin.shape
    G, K, dg = codebooks.shape
    t_blk = 1024 if T % 1024 == 0 else T
    grid = (B, T // t_blk)

    zq, codes, loss = pl.pallas_call(
        functools.partial(_vq_kernel, n_total=B * C * T),
        grid=grid,
        in_specs=[
            pl.BlockSpec((1, C, t_blk), lambda b, t: (b, 0, t)),
            pl.BlockSpec((G, K, dg), lambda b, t: (0, 0, 0)),
        ],
        out_specs=[
            pl.BlockSpec((1, C, t_blk), lambda b, t: (b, 0, t)),
            pl.BlockSpec((1, G, t_blk), lambda b, t: (b, 0, t)),
            pl.BlockSpec((1, 1), lambda b, t: (0, 0),
                         memory_space=pltpu.SMEM),
        ],
        out_shape=[
            jax.ShapeDtypeStruct((B, C, T), jnp.float32),
            jax.ShapeDtypeStruct((B, G, T), jnp.int32),
            jax.ShapeDtypeStruct((1, 1), jnp.float32),
        ],
    )(xin, codebooks)
    return zq, loss[0, 0], codes
